# baseline (speedup 1.0000x reference)
import jax
import jax.numpy as jnp
from jax.experimental import pallas as pl
from jax.experimental.pallas import tpu as pltpu

_E = 64
_R = 256
_DI = 2048
_DO = 2048
_T = 128
_EPS = 2
_STEPS = _E // _EPS
_WB_STEPS = 16
_WB_BLK = _DO // _WB_STEPS
_HALF = _EPS // 2


def _moe_kernel(x_ref, wb_ref, bb_ref, a_ref, wr_ref, b0_ref, b1_ref,
                out_ref, xb_ref, hw_ref):
    e = pl.program_id(0)

    @pl.when(e == 0)
    def _prologue():
        x = x_ref[...]
        xb_ref[...] = x.astype(jnp.bfloat16)
        h = jax.lax.dot_general(
            x, a_ref[...], (((1,), (1,)), ((), ())),
            preferred_element_type=jnp.float32)
        logits = jax.lax.dot_general(
            x, wr_ref[...], (((1,), (1,)), ((), ())),
            preferred_element_type=jnp.float32)
        p = jax.nn.softmax(logits, axis=-1)
        cols = jax.lax.broadcasted_iota(jnp.int32, p.shape, 1)
        m1 = jnp.max(p, axis=-1, keepdims=True)
        i1 = jnp.min(jnp.where(p == m1, cols, _E), axis=-1, keepdims=True)
        p2 = jnp.where(cols == i1, -1.0, p)
        m2 = jnp.max(p2, axis=-1, keepdims=True)
        i2 = jnp.min(jnp.where(p2 == m2, cols, _E), axis=-1, keepdims=True)
        mask = (cols == i1) | (cols == i2)
        denom = m1 + m2 + 1e-6
        pw = jnp.where(mask, p / denom, 0.0)
        pwt = pw.T
        hw_ref[...] = (pwt[:, :, None] * h[None, :, :]).astype(jnp.bfloat16)
        out_ref[...] = jnp.broadcast_to(bb_ref[...], out_ref.shape)

    @pl.when(e < _WB_STEPS)
    def _base():
        base_blk = jax.lax.dot_general(
            xb_ref[...], wb_ref[...].astype(jnp.bfloat16),
            (((1,), (1,)), ((), ())),
            preferred_element_type=jnp.float32)
        out_ref[:, pl.ds(e * _WB_BLK, _WB_BLK)] += base_blk

    acc = None
    for j in range(_EPS):
        b_ref = b0_ref if j < _HALF else b1_ref
        hw = hw_ref[e * _EPS + j]
        dj = jax.lax.dot_general(
            hw, b_ref[j % _HALF].astype(jnp.bfloat16),
            (((1,), (1,)), ((), ())),
            preferred_element_type=jnp.float32)
        acc = dj if acc is None else acc + dj
    out_ref[...] += acc


def kernel(x, W_base, b_base, A, B, W_router):
    x2 = x.reshape(_T, _DI)
    bb2 = b_base.reshape(1, _DO)
    out = pl.pallas_call(
        _moe_kernel,
        grid=(_STEPS,),
        in_specs=[
            pl.BlockSpec((_T, _DI), lambda e: (0, 0)),
            pl.BlockSpec((_WB_BLK, _DI), lambda e: (jnp.minimum(e, _WB_STEPS - 1), 0)),
            pl.BlockSpec((1, _DO), lambda e: (0, 0)),
            pl.BlockSpec((_R, _DI), lambda e: (0, 0)),
            pl.BlockSpec((_E, _DI), lambda e: (0, 0)),
            pl.BlockSpec((_HALF, _DO, _R), lambda e: (2 * e, 0, 0)),
            pl.BlockSpec((_HALF, _DO, _R), lambda e: (2 * e + 1, 0, 0)),
        ],
        out_specs=pl.BlockSpec((_T, _DO), lambda e: (0, 0)),
        out_shape=jax.ShapeDtypeStruct((_T, _DO), jnp.float32),
        scratch_shapes=[
            pltpu.VMEM((_T, _DI), jnp.bfloat16),
            pltpu.VMEM((_E, _T, _R), jnp.bfloat16),
        ],
        compiler_params=pltpu.CompilerParams(
            dimension_semantics=("arbitrary",),
        ),
    )(x2, W_base, bb2, A, W_router, B, B)
    return out.reshape(_T, 1, _DO)


# speedup vs baseline: 1.1282x; 1.1282x over previous
import jax
import jax.numpy as jnp
from jax.experimental import pallas as pl
from jax.experimental.pallas import tpu as pltpu

_E = 64
_R = 256
_DI = 2048
_DO = 2048
_T = 128
_EPS = 8
_STEPS = _E // _EPS
_WB_BLK = _DO // _STEPS
_HALF = _EPS // 2


def _moe_kernel(x_ref, wb_ref, bb_ref, a_ref, wr_ref, b0_ref, b1_ref,
                out_ref, xb_ref, hw_ref):
    e = pl.program_id(0)

    @pl.when(e == 0)
    def _prologue():
        x = x_ref[...]
        xb_ref[...] = x.astype(jnp.bfloat16)
        h = jax.lax.dot_general(
            x, a_ref[...], (((1,), (1,)), ((), ())),
            preferred_element_type=jnp.float32)
        logits = jax.lax.dot_general(
            x, wr_ref[...], (((1,), (1,)), ((), ())),
            preferred_element_type=jnp.float32)
        p = jax.nn.softmax(logits, axis=-1)
        cols = jax.lax.broadcasted_iota(jnp.int32, p.shape, 1)
        m1 = jnp.max(p, axis=-1, keepdims=True)
        i1 = jnp.min(jnp.where(p == m1, cols, _E), axis=-1, keepdims=True)
        p2 = jnp.where(cols == i1, -1.0, p)
        m2 = jnp.max(p2, axis=-1, keepdims=True)
        i2 = jnp.min(jnp.where(p2 == m2, cols, _E), axis=-1, keepdims=True)
        mask = (cols == i1) | (cols == i2)
        denom = m1 + m2 + 1e-6
        pw = jnp.where(mask, p / denom, 0.0)
        pwt = pw.T
        hw_ref[...] = (pwt[:, :, None] * h[None, :, :]).astype(jnp.bfloat16)
        out_ref[...] = jnp.broadcast_to(bb_ref[...], out_ref.shape)

    base_blk = jax.lax.dot_general(
        xb_ref[...], wb_ref[...].astype(jnp.bfloat16),
        (((1,), (1,)), ((), ())),
        preferred_element_type=jnp.float32)
    out_ref[:, pl.ds(e * _WB_BLK, _WB_BLK)] += base_blk

    acc = None
    for j in range(_EPS):
        b_ref = b0_ref if j < _HALF else b1_ref
        hw = hw_ref[e * _EPS + j]
        dj = jax.lax.dot_general(
            hw, b_ref[j % _HALF].astype(jnp.bfloat16),
            (((1,), (1,)), ((), ())),
            preferred_element_type=jnp.float32)
        acc = dj if acc is None else acc + dj
    out_ref[...] += acc


def kernel(x, W_base, b_base, A, B, W_router):
    x2 = x.reshape(_T, _DI)
    bb2 = b_base.reshape(1, _DO)
    out = pl.pallas_call(
        _moe_kernel,
        grid=(_STEPS,),
        in_specs=[
            pl.BlockSpec((_T, _DI), lambda e: (0, 0)),
            pl.BlockSpec((_WB_BLK, _DI), lambda e: (e, 0)),
            pl.BlockSpec((1, _DO), lambda e: (0, 0)),
            pl.BlockSpec((_R, _DI), lambda e: (0, 0)),
            pl.BlockSpec((_E, _DI), lambda e: (0, 0)),
            pl.BlockSpec((_HALF, _DO, _R), lambda e: (2 * e, 0, 0)),
            pl.BlockSpec((_HALF, _DO, _R), lambda e: (2 * e + 1, 0, 0)),
        ],
        out_specs=pl.BlockSpec((_T, _DO), lambda e: (0, 0)),
        out_shape=jax.ShapeDtypeStruct((_T, _DO), jnp.float32),
        scratch_shapes=[
            pltpu.VMEM((_T, _DI), jnp.bfloat16),
            pltpu.VMEM((_E, _T, _R), jnp.bfloat16),
        ],
        compiler_params=pltpu.CompilerParams(
            dimension_semantics=("arbitrary",),
        ),
    )(x2, W_base, bb2, A, W_router, B, B)
    return out.reshape(_T, 1, _DO)


# speedup vs baseline: 1.1547x; 1.0235x over previous
import jax
import jax.numpy as jnp
from jax.experimental import pallas as pl
from jax.experimental.pallas import tpu as pltpu

_E = 64
_R = 256
_DI = 2048
_DO = 2048
_T = 128
_EPS = 4
_STEPS = _E // _EPS
_WB_BLK = _DO // _STEPS
_HALF = _EPS // 2


def _moe_kernel(x_ref, wb_ref, bb_ref, a_ref, wr_ref, b0_ref, b1_ref,
                out_ref, xb_ref, hw_ref):
    e = pl.program_id(0)

    @pl.when(e == 0)
    def _prologue():
        x = x_ref[...]
        xb_ref[...] = x.astype(jnp.bfloat16)
        h = jax.lax.dot_general(
            x, a_ref[...], (((1,), (1,)), ((), ())),
            preferred_element_type=jnp.float32)
        logits = jax.lax.dot_general(
            x, wr_ref[...], (((1,), (1,)), ((), ())),
            preferred_element_type=jnp.float32)
        p = jax.nn.softmax(logits, axis=-1)
        cols = jax.lax.broadcasted_iota(jnp.int32, p.shape, 1)
        m1 = jnp.max(p, axis=-1, keepdims=True)
        i1 = jnp.min(jnp.where(p == m1, cols, _E), axis=-1, keepdims=True)
        p2 = jnp.where(cols == i1, -1.0, p)
        m2 = jnp.max(p2, axis=-1, keepdims=True)
        i2 = jnp.min(jnp.where(p2 == m2, cols, _E), axis=-1, keepdims=True)
        mask = (cols == i1) | (cols == i2)
        denom = m1 + m2 + 1e-6
        pw = jnp.where(mask, p / denom, 0.0)
        pwt = pw.T
        hw_ref[...] = (pwt[:, :, None] * h[None, :, :]).astype(jnp.bfloat16)
        out_ref[...] = jnp.broadcast_to(bb_ref[...], out_ref.shape)

    base_blk = jax.lax.dot_general(
        xb_ref[...], wb_ref[...].astype(jnp.bfloat16),
        (((1,), (1,)), ((), ())),
        preferred_element_type=jnp.float32)
    out_ref[:, pl.ds(e * _WB_BLK, _WB_BLK)] += base_blk

    acc = None
    for j in range(_EPS):
        b_ref = b0_ref if j < _HALF else b1_ref
        hw = hw_ref[e * _EPS + j]
        dj = jax.lax.dot_general(
            hw, b_ref[j % _HALF].astype(jnp.bfloat16),
            (((1,), (1,)), ((), ())),
            preferred_element_type=jnp.float32)
        acc = dj if acc is None else acc + dj
    out_ref[...] += acc


def kernel(x, W_base, b_base, A, B, W_router):
    x2 = x.reshape(_T, _DI)
    bb2 = b_base.reshape(1, _DO)
    out = pl.pallas_call(
        _moe_kernel,
        grid=(_STEPS,),
        in_specs=[
            pl.BlockSpec((_T, _DI), lambda e: (0, 0)),
            pl.BlockSpec((_WB_BLK, _DI), lambda e: (e, 0)),
            pl.BlockSpec((1, _DO), lambda e: (0, 0)),
            pl.BlockSpec((_R, _DI), lambda e: (0, 0)),
            pl.BlockSpec((_E, _DI), lambda e: (0, 0)),
            pl.BlockSpec((_HALF, _DO, _R), lambda e: (2 * e, 0, 0)),
            pl.BlockSpec((_HALF, _DO, _R), lambda e: (2 * e + 1, 0, 0)),
        ],
        out_specs=pl.BlockSpec((_T, _DO), lambda e: (0, 0)),
        out_shape=jax.ShapeDtypeStruct((_T, _DO), jnp.float32),
        scratch_shapes=[
            pltpu.VMEM((_T, _DI), jnp.bfloat16),
            pltpu.VMEM((_E, _T, _R), jnp.bfloat16),
        ],
        compiler_params=pltpu.CompilerParams(
            dimension_semantics=("arbitrary",),
        ),
    )(x2, W_base, bb2, A, W_router, B, B)
    return out.reshape(_T, 1, _DO)
